# Initial kernel scaffold; baseline (speedup 1.0000x reference)
import jax
import jax.numpy as jnp
from jax.experimental import pallas as pl


def kernel(x, table, W, b):
    raise NotImplementedError("write your pallas kernel here")


# speedup vs baseline: 2.2662x; 2.2662x over previous
import functools

import jax
import jax.numpy as jnp
from jax import lax
from jax.experimental import pallas as pl
from jax.experimental.pallas import tpu as pltpu
from jax.experimental.pallas import tpu_sc as plsc

_N_ITEMS = 1000000
_H = 64
_ROWS_PER_BLK = 8000


def _matvec_body(table_ref, w_ref, b_ref, out_ref):
    t = table_ref[...]
    w = w_ref[...]
    out_ref[...] = jnp.dot(t, w, preferred_element_type=jnp.float32) + b_ref[0, 0]


def _table_matvec(table, W, b):
    nblk = _N_ITEMS // _ROWS_PER_BLK
    return pl.pallas_call(
        _matvec_body,
        grid=(nblk,),
        in_specs=[
            pl.BlockSpec((_ROWS_PER_BLK, _H), lambda i: (i, 0)),
            pl.BlockSpec((_H, 1), lambda i: (0, 0)),
            pl.BlockSpec((1, 1), lambda i: (0, 0)),
        ],
        out_specs=pl.BlockSpec((_ROWS_PER_BLK, 1), lambda i: (i, 0)),
        out_shape=jax.ShapeDtypeStruct((_N_ITEMS, 1), jnp.float32),
    )(table, W, b.reshape(1, 1))


def _make_gather(n_idx):
    nw = 32
    assert n_idx % (8 * nw) == 0
    per_w = n_idx // nw
    mesh = plsc.VectorSubcoreMesh(core_axis_name="c", subcore_axis_name="s")

    @functools.partial(
        pl.kernel,
        mesh=mesh,
        out_type=jax.ShapeDtypeStruct((n_idx,), jnp.float32),
        scratch_types=[
            pltpu.VMEM((per_w,), jnp.int32),
            pltpu.VMEM((per_w,), jnp.float32),
            pltpu.SemaphoreType.DMA,
        ],
    )
    def _gather(tw_hbm, idx_hbm, out_hbm, idx_v, val_v, sem):
        wid = lax.axis_index("s") * 2 + lax.axis_index("c")
        base = wid * per_w
        pltpu.sync_copy(idx_hbm.at[pl.ds(base, per_w)], idx_v)
        pltpu.async_copy(tw_hbm.at[idx_v], val_v, sem).wait()
        pltpu.sync_copy(val_v, out_hbm.at[pl.ds(base, per_w)])

    return _gather


def kernel(x, table, W, b):
    bsz, hist = x.shape
    tw = _table_matvec(table, W, b).reshape(-1)
    xf = x.reshape(-1).astype(jnp.int32)
    out = _make_gather(bsz * hist)(tw, xf)
    return out.reshape(bsz, hist, 1)


# speedup vs baseline: 3.2565x; 1.4370x over previous
import functools

import jax
import jax.numpy as jnp
from jax import lax
from jax.experimental import pallas as pl
from jax.experimental.pallas import tpu as pltpu
from jax.experimental.pallas import tpu_sc as plsc

_N_ITEMS = 1000000
_H = 64
_ROWS_PER_BLK = 20000


def _matvec_body(wt_ref, table_ref, b_ref, out_ref):
    out_ref[0] = lax.dot_general(
        wt_ref[...], table_ref[...],
        dimension_numbers=(((1,), (1,)), ((), ())),
        preferred_element_type=jnp.float32,
    ) + b_ref[0, 0]


def _table_matvec(table, W, b):
    nblk = _N_ITEMS // _ROWS_PER_BLK
    out = pl.pallas_call(
        _matvec_body,
        grid=(nblk,),
        in_specs=[
            pl.BlockSpec((1, _H), lambda i: (0, 0)),
            pl.BlockSpec((_ROWS_PER_BLK, _H), lambda i: (i, 0)),
            pl.BlockSpec((1, 1), lambda i: (0, 0)),
        ],
        out_specs=pl.BlockSpec((1, 1, _ROWS_PER_BLK), lambda i: (i, 0, 0)),
        out_shape=jax.ShapeDtypeStruct((nblk, 1, _ROWS_PER_BLK), jnp.float32),
    )(W.reshape(1, _H), table, b.reshape(1, 1))
    return out


def _make_gather(n_idx):
    nw = 32
    assert n_idx % (8 * nw) == 0
    per_w = n_idx // nw
    mesh = plsc.VectorSubcoreMesh(core_axis_name="c", subcore_axis_name="s")

    @functools.partial(
        pl.kernel,
        mesh=mesh,
        out_type=jax.ShapeDtypeStruct((n_idx,), jnp.float32),
        scratch_types=[
            pltpu.VMEM((per_w,), jnp.int32),
            pltpu.VMEM((per_w,), jnp.float32),
            pltpu.SemaphoreType.DMA,
        ],
    )
    def _gather(tw_hbm, idx_hbm, out_hbm, idx_v, val_v, sem):
        wid = lax.axis_index("s") * 2 + lax.axis_index("c")
        base = wid * per_w
        pltpu.sync_copy(idx_hbm.at[pl.ds(base, per_w)], idx_v)
        pltpu.async_copy(tw_hbm.at[idx_v], val_v, sem).wait()
        pltpu.sync_copy(val_v, out_hbm.at[pl.ds(base, per_w)])

    return _gather


def kernel(x, table, W, b):
    bsz, hist = x.shape
    tw = _table_matvec(table, W, b).reshape(-1)
    xf = x.reshape(-1).astype(jnp.int32)
    out = _make_gather(bsz * hist)(tw, xf)
    return out.reshape(bsz, hist, 1)


# speedup vs baseline: 3.3005x; 1.0135x over previous
import functools

import jax
import jax.numpy as jnp
from jax import lax
from jax.experimental import pallas as pl
from jax.experimental.pallas import tpu as pltpu
from jax.experimental.pallas import tpu_sc as plsc

_N_ITEMS = 1000000
_H = 64
_N_STREAMS = 8
_SUB_ROWS = 5000
_ROWS_PER_BLK = _N_STREAMS * _SUB_ROWS


def _matvec_body(*refs):
    wt_ref = refs[0]
    t_refs = refs[1:1 + _N_STREAMS]
    b_ref = refs[1 + _N_STREAMS]
    out_ref = refs[2 + _N_STREAMS]
    wt = wt_ref[...]
    bias = b_ref[0, 0]
    for k, tr in enumerate(t_refs):
        out_ref[0, :, pl.ds(k * _SUB_ROWS, _SUB_ROWS)] = lax.dot_general(
            wt, tr[...],
            dimension_numbers=(((1,), (1,)), ((), ())),
            preferred_element_type=jnp.float32,
        ) + bias


def _table_matvec(table, W, b):
    nblk = _N_ITEMS // _ROWS_PER_BLK
    t_specs = [
        pl.BlockSpec((_SUB_ROWS, _H), lambda i, k=k: (_N_STREAMS * i + k, 0))
        for k in range(_N_STREAMS)
    ]
    out = pl.pallas_call(
        _matvec_body,
        grid=(nblk,),
        in_specs=[pl.BlockSpec((1, _H), lambda i: (0, 0))]
        + t_specs
        + [pl.BlockSpec((1, 1), lambda i: (0, 0))],
        out_specs=pl.BlockSpec((1, 1, _ROWS_PER_BLK), lambda i: (i, 0, 0)),
        out_shape=jax.ShapeDtypeStruct((nblk, 1, _ROWS_PER_BLK), jnp.float32),
    )(W.reshape(1, _H), *([table] * _N_STREAMS), b.reshape(1, 1))
    return out


def _make_gather(n_idx):
    nw = 32
    assert n_idx % (8 * nw) == 0
    per_w = n_idx // nw
    mesh = plsc.VectorSubcoreMesh(core_axis_name="c", subcore_axis_name="s")

    @functools.partial(
        pl.kernel,
        mesh=mesh,
        out_type=jax.ShapeDtypeStruct((n_idx,), jnp.float32),
        scratch_types=[
            pltpu.VMEM((per_w,), jnp.int32),
            pltpu.VMEM((per_w,), jnp.float32),
            pltpu.SemaphoreType.DMA,
        ],
    )
    def _gather(tw_hbm, idx_hbm, out_hbm, idx_v, val_v, sem):
        wid = lax.axis_index("s") * 2 + lax.axis_index("c")
        base = wid * per_w
        pltpu.sync_copy(idx_hbm.at[pl.ds(base, per_w)], idx_v)
        pltpu.async_copy(tw_hbm.at[idx_v], val_v, sem).wait()
        pltpu.sync_copy(val_v, out_hbm.at[pl.ds(base, per_w)])

    return _gather


def kernel(x, table, W, b):
    bsz, hist = x.shape
    tw = _table_matvec(table, W, b).reshape(-1)
    xf = x.reshape(-1).astype(jnp.int32)
    out = _make_gather(bsz * hist)(tw, xf)
    return out.reshape(bsz, hist, 1)


# speedup vs baseline: 15.4316x; 4.6756x over previous
import functools

import jax
import jax.numpy as jnp
from jax import lax
from jax.experimental import pallas as pl
from jax.experimental.pallas import tpu as pltpu
from jax.experimental.pallas import tpu_sc as plsc

_N_ITEMS = 1000000
_H = 64
_COLS_PER_BLK = 40960


def _matvec_body(wt_ref, tt_ref, b_ref, out_ref):
    res = jnp.dot(wt_ref[...], tt_ref[...], preferred_element_type=jnp.float32)
    out_ref[...] = res[0] + b_ref[0, 0]


def _table_matvec(tableT, W, b):
    nblk = pl.cdiv(_N_ITEMS, _COLS_PER_BLK)
    out = pl.pallas_call(
        _matvec_body,
        grid=(nblk,),
        in_specs=[
            pl.BlockSpec((1, _H), lambda i: (0, 0)),
            pl.BlockSpec((_H, _COLS_PER_BLK), lambda i: (0, i)),
            pl.BlockSpec((1, 1), lambda i: (0, 0)),
        ],
        out_specs=pl.BlockSpec((_COLS_PER_BLK,), lambda i: (i,)),
        out_shape=jax.ShapeDtypeStruct((_N_ITEMS,), jnp.float32),
    )(W.reshape(1, _H), tableT, b.reshape(1, 1))
    return out


def _make_gather(n_idx):
    nw = 32
    assert n_idx % (8 * nw) == 0
    per_w = n_idx // nw
    mesh = plsc.VectorSubcoreMesh(core_axis_name="c", subcore_axis_name="s")

    @functools.partial(
        pl.kernel,
        mesh=mesh,
        out_type=jax.ShapeDtypeStruct((n_idx,), jnp.float32),
        scratch_types=[
            pltpu.VMEM((per_w,), jnp.int32),
            pltpu.VMEM((per_w,), jnp.float32),
            pltpu.SemaphoreType.DMA,
        ],
    )
    def _gather(tw_hbm, idx_hbm, out_hbm, idx_v, val_v, sem):
        wid = lax.axis_index("s") * 2 + lax.axis_index("c")
        base = wid * per_w
        pltpu.sync_copy(idx_hbm.at[pl.ds(base, per_w)], idx_v)
        pltpu.async_copy(tw_hbm.at[idx_v], val_v, sem).wait()
        pltpu.sync_copy(val_v, out_hbm.at[pl.ds(base, per_w)])

    return _gather


def kernel(x, table, W, b):
    bsz, hist = x.shape
    tw = _table_matvec(table.T, W, b)
    xf = x.T.reshape(-1).astype(jnp.int32)
    out = _make_gather(bsz * hist)(tw, xf)
    return out.reshape(hist, bsz, 1).transpose(1, 0, 2)
